# Initial kernel scaffold; baseline (speedup 1.0000x reference)
import jax
import jax.numpy as jnp
from jax.experimental import pallas as pl


def kernel(seq_len, bias_table):
    raise NotImplementedError("write your pallas kernel here")


# speedup vs baseline: 37.0543x; 37.0543x over previous
import jax
import jax.numpy as jnp
from jax import lax
from jax.experimental import pallas as pl
from jax.experimental.pallas import tpu as pltpu
from jax.experimental.pallas import tpu_sc as plsc

S = 2048
H = 16
COL = 2 * S * 2
NC = 2
NS = 16


def _bias_body(table_hbm, out_hbm, shifts_ref):
    c = lax.axis_index("c")
    s = lax.axis_index("s")
    w = s * NC + c
    h = w // 2
    base = (w % 2) * (S // 2)

    pltpu.sync_copy(table_hbm.at[pl.ds(h * (8 * COL), 8 * COL)], shifts_ref)

    def row(t, carry):
        i = base + t
        start = (2 * S - 1) - i
        sh = lax.rem(start, 8)
        off = pl.multiple_of(sh * COL + (start - sh), 8)
        pltpu.sync_copy(shifts_ref.at[pl.ds(off, S)],
                        out_hbm.at[pl.ds((h * S + i) * S, S)])
        return carry

    lax.fori_loop(0, S // 2, row, 0)


def kernel(seq_len, bias_table):
    del seq_len
    colpad = jnp.pad(jnp.transpose(bias_table), ((0, 0), (0, COL + 8 - bias_table.shape[0])))
    tab = jnp.stack([colpad[:, sh:sh + COL] for sh in range(8)], axis=1)
    tab = tab.reshape(H * 8 * COL)
    run = pl.kernel(
        _bias_body,
        out_type=jax.ShapeDtypeStruct((H * S * S,), jnp.float32),
        mesh=plsc.VectorSubcoreMesh(core_axis_name="c", subcore_axis_name="s"),
        scratch_types=[pltpu.VMEM((8 * COL,), jnp.float32)],
    )
    return run(tab).reshape(1, H, S, S)


# speedup vs baseline: 69.1379x; 1.8659x over previous
import jax
import jax.numpy as jnp
from jax import lax
from jax.experimental import pallas as pl
from jax.experimental.pallas import tpu as pltpu
from jax.experimental.pallas import tpu_sc as plsc

S = 2048
H = 16
LP = 6144
NC = 2
TI = S // 8
TJ = S // 128


def _bias_body(table_hbm, out_hbm, b2_ref):
    c = lax.axis_index("c")
    s = lax.axis_index("s")
    w = s * NC + c
    h = w // 2
    base_i = (w % 2) * (TI // 2)

    for sh in range(8):
        pltpu.sync_copy(table_hbm.at[pl.ds((h * 8 + sh) * LP, LP)],
                        b2_ref.at[sh])

    def tile(t, carry):
        ti = base_i + t // TJ
        tj = lax.rem(t, TJ)
        base7 = pl.multiple_of((4095 - 7) - 8 * ti + 128 * tj, 8)
        pltpu.sync_copy(b2_ref.at[:, pl.ds(base7, 128)],
                        out_hbm.at[h, ti, tj])
        return carry

    lax.fori_loop(0, (TI // 2) * TJ, tile, 0)


def kernel(seq_len, bias_table):
    del seq_len
    col = jnp.transpose(bias_table)
    tab = jnp.stack([col[:, 7 - sh:7 - sh + LP] for sh in range(8)], axis=1)
    tab = tab.reshape(H * 8 * LP)
    run = pl.kernel(
        _bias_body,
        out_type=jax.ShapeDtypeStruct((H, TI, TJ, 8, 128), jnp.float32),
        mesh=plsc.VectorSubcoreMesh(core_axis_name="c", subcore_axis_name="s"),
        scratch_types=[pltpu.VMEM((8, LP), jnp.float32)],
        compiler_params=pltpu.CompilerParams(use_tc_tiling_on_sc=False),
    )
    out5 = run(tab)
    return out5.transpose(0, 1, 3, 2, 4).reshape(1, H, S, S)


# speedup vs baseline: 139.4646x; 2.0172x over previous
import jax
import jax.numpy as jnp
from jax import lax
from jax.experimental import pallas as pl
from jax.experimental.pallas import tpu as pltpu
from jax.experimental.pallas import tpu_sc as plsc

S = 2048
H = 16
LP = 6144
NC = 2
TI = S // 8
TJ = S // 128


def _bias_body(table_hbm, out_hbm, b2_ref, sem):
    c = lax.axis_index("c")
    s = lax.axis_index("s")
    w = s * NC + c
    h = w // 2
    base_i = (w % 2) * (TI // 2)

    for sh in range(8):
        pltpu.sync_copy(table_hbm.at[pl.ds((h * 8 + sh) * LP, LP)],
                        b2_ref.at[sh])

    LAG = 4
    NW = TI // 2

    def row_dmas(v):
        ti = base_i + v
        base0 = pl.multiple_of((4095 - 7) - 8 * ti, 8)
        return [pltpu.make_async_copy(
                    b2_ref.at[:, pl.ds(base0 + 128 * tj, 128)],
                    out_hbm.at[h, ti, tj], sem)
                for tj in range(TJ)]

    def wave(v, carry):
        @pl.when(v < NW)
        def _():
            for d in row_dmas(v):
                d.start()

        @pl.when(v >= LAG)
        def _():
            for d in row_dmas(v - LAG):
                d.wait()
        return carry

    lax.fori_loop(0, NW + LAG, wave, 0)


def kernel(seq_len, bias_table):
    del seq_len
    col = jnp.transpose(bias_table)
    tab = jnp.stack([col[:, 7 - sh:7 - sh + LP] for sh in range(8)], axis=1)
    tab = tab.reshape(H * 8 * LP)
    run = pl.kernel(
        _bias_body,
        out_type=jax.ShapeDtypeStruct((H, TI, TJ, 8, 128), jnp.float32),
        mesh=plsc.VectorSubcoreMesh(core_axis_name="c", subcore_axis_name="s"),
        scratch_types=[pltpu.VMEM((8, LP), jnp.float32),
                       pltpu.SemaphoreType.DMA],
        compiler_params=pltpu.CompilerParams(use_tc_tiling_on_sc=False),
    )
    out5 = run(tab)
    return out5.transpose(0, 1, 3, 2, 4).reshape(1, H, S, S)


# speedup vs baseline: 140.5706x; 1.0079x over previous
import jax
import jax.numpy as jnp
from jax import lax
from jax.experimental import pallas as pl
from jax.experimental.pallas import tpu as pltpu
from jax.experimental.pallas import tpu_sc as plsc

S = 2048
H = 16
LP = 6144
NC = 2
TI = S // 8
TJ = S // 128


def _bias_body(table_hbm, out_hbm, b2_ref, sem):
    c = lax.axis_index("c")
    s = lax.axis_index("s")
    w = s * NC + c
    h = w // 2
    base_i = (w % 2) * (TI // 2)

    for sh in range(8):
        pltpu.sync_copy(table_hbm.at[pl.ds((h * 8 + sh) * LP, LP)],
                        b2_ref.at[sh])

    LAG = 8
    NW = TI // 2

    def row_dmas(v):
        ti = base_i + v
        base0 = pl.multiple_of((4095 - 7) - 8 * ti, 8)
        return [pltpu.make_async_copy(
                    b2_ref.at[:, pl.ds(base0 + 128 * tj, 128)],
                    out_hbm.at[h, ti, tj], sem)
                for tj in range(TJ)]

    def wave(v, carry):
        @pl.when(v < NW)
        def _():
            for d in row_dmas(v):
                d.start()

        @pl.when(v >= LAG)
        def _():
            for d in row_dmas(v - LAG):
                d.wait()
        return carry

    lax.fori_loop(0, NW + LAG, wave, 0)


def kernel(seq_len, bias_table):
    del seq_len
    col = jnp.transpose(bias_table)
    tab = jnp.stack([col[:, 7 - sh:7 - sh + LP] for sh in range(8)], axis=1)
    tab = tab.reshape(H * 8 * LP)
    run = pl.kernel(
        _bias_body,
        out_type=jax.ShapeDtypeStruct((H, TI, TJ, 8, 128), jnp.float32),
        mesh=plsc.VectorSubcoreMesh(core_axis_name="c", subcore_axis_name="s"),
        scratch_types=[pltpu.VMEM((8, LP), jnp.float32),
                       pltpu.SemaphoreType.DMA],
        compiler_params=pltpu.CompilerParams(use_tc_tiling_on_sc=False),
    )
    out5 = run(tab)
    return out5.transpose(0, 1, 3, 2, 4).reshape(1, H, S, S)
